# baseline (speedup 1.0000x reference)
import functools

import jax
import jax.numpy as jnp
from jax import lax
from jax.experimental import pallas as pl
from jax.experimental.pallas import tpu as pltpu
from jax.experimental.pallas import tpu_sc as plsc


def _sc_gather_t(table_t, ids_2d, n, r, chunk):
    info = plsc.get_sparse_core_info()
    nc, ns = info.num_cores, info.num_subcores
    nw = nc * ns
    n_per_w = n // nw
    n_chunks = n_per_w // chunk

    mesh = plsc.VectorSubcoreMesh(core_axis_name="c", subcore_axis_name="s")

    @functools.partial(
        pl.kernel,
        mesh=mesh,
        out_type=jax.ShapeDtypeStruct((r, n), jnp.float32),
        scratch_types=[
            pltpu.VMEM((n_chunks, chunk), jnp.int32),
            pltpu.VMEM((r, n_per_w), jnp.float32),
            pltpu.SemaphoreType.DMA,
            pltpu.SemaphoreType.DMA,
        ],
        compiler_params=pltpu.CompilerParams(use_tc_tiling_on_sc=False),
    )
    def gather_cols(table_hbm, idx_hbm, out_hbm, idx_v, cols_v, sem, osem):
        wid = lax.axis_index("s") * nc + lax.axis_index("c")
        base = wid * n_per_w
        pltpu.sync_copy(idx_hbm.at[pl.ds(wid * n_chunks, n_chunks)], idx_v)
        for dim in range(r):
            row = table_hbm.at[dim]
            for jc in range(n_chunks):
                pltpu.async_copy(
                    row.at[idx_v.at[jc]],
                    cols_v.at[dim, pl.ds(jc * chunk, chunk)],
                    sem,
                )
        out_slice = out_hbm.at[:, pl.ds(base, n_per_w)]
        pltpu.make_async_copy(out_slice, cols_v, sem).wait()
        pltpu.async_copy(cols_v, out_slice, osem).wait()

    return gather_cols(table_t, ids_2d)


def kernel(input_ids, input_states, lora_A, lora_B_w):
    b, s = input_ids.shape
    h = input_states.shape[-1]
    r = lora_A.shape[1]
    n = b * s
    chunk = 128

    ids_2d = input_ids.reshape(n // chunk, chunk).astype(jnp.int32)
    gathered_t = _sc_gather_t(lora_A.T, ids_2d, n, r, chunk)

    x2d = input_states.reshape(n, h)
    w_t = lora_B_w.T
    blk = 512

    def tc_body(g_ref, x_ref, w_ref, o_ref):
        prj = lax.dot_general(
            g_ref[...],
            w_ref[...],
            dimension_numbers=(((0,), (0,)), ((), ())),
            preferred_element_type=jnp.float32,
        )
        o_ref[...] = x_ref[...] + prj

    out2d = pl.pallas_call(
        tc_body,
        grid=(n // blk,),
        in_specs=[
            pl.BlockSpec((r, blk), lambda i: (0, i)),
            pl.BlockSpec((blk, h), lambda i: (i, 0)),
            pl.BlockSpec((r, h), lambda i: (0, 0)),
        ],
        out_specs=pl.BlockSpec((blk, h), lambda i: (i, 0)),
        out_shape=jax.ShapeDtypeStruct((n, h), jnp.float32),
    )(gathered_t, x2d, w_t)

    return out2d.reshape(b, s, h)


# speedup vs baseline: 12.9391x; 12.9391x over previous
import functools

import jax
import jax.numpy as jnp
from jax import lax
from jax.experimental import pallas as pl
from jax.experimental.pallas import tpu as pltpu
from jax.experimental.pallas import tpu_sc as plsc


def _sc_gather_t(table_t, ids, n, r):
    info = plsc.get_sparse_core_info()
    nc, ns = info.num_cores, info.num_subcores
    nw = nc * ns
    n_per_w = n // nw
    cpc = 16
    n_chunks = n_per_w // cpc

    mesh = plsc.VectorSubcoreMesh(core_axis_name="c", subcore_axis_name="s")

    @functools.partial(
        pl.kernel,
        mesh=mesh,
        out_type=jax.ShapeDtypeStruct((r, n), jnp.float32),
        scratch_types=[
            pltpu.VMEM((n_per_w,), jnp.int32),
            pltpu.VMEM((r, cpc * 128), jnp.float32),
            pltpu.VMEM((r, n_per_w), jnp.float32),
            pltpu.SemaphoreType.DMA,
            pltpu.SemaphoreType.DMA,
        ],
        compiler_params=pltpu.CompilerParams(needs_layout_passes=False),
    )
    def gather_cols(table_hbm, idx_hbm, out_hbm, idx_v, buf, cols_v, sem, osem):
        wid = lax.axis_index("s") * nc + lax.axis_index("c")
        base = wid * n_per_w
        pltpu.sync_copy(idx_hbm.at[pl.ds(base, n_per_w)], idx_v)
        lanes = jnp.arange(16, dtype=jnp.int32)

        def body(jc, _):
            vec = idx_v[pl.ds(jc * cpc, 16)]
            for k in range(cpc):
                rid = vec[k]
                off = pl.multiple_of((rid // 128) * 128, 128)
                pltpu.async_copy(
                    table_hbm.at[:, pl.ds(off, 128)],
                    buf.at[:, pl.ds(k * 128, 128)],
                    sem,
                )
            pltpu.make_async_copy(
                table_hbm.at[:, pl.ds(0, cpc * 128)], buf, sem
            ).wait()
            for k in range(cpc):
                rid = vec[k]
                col = (rid % 128) + k * 128
                v = plsc.load_gather(buf, [lanes, jnp.broadcast_to(col, (16,))])
                j = jc * cpc + k
                plsc.store_scatter(
                    cols_v, [lanes, jnp.broadcast_to(j, (16,))], v
                )
            return 0

        lax.fori_loop(0, n_chunks, body, 0)
        pltpu.async_copy(cols_v, out_hbm.at[:, pl.ds(base, n_per_w)], osem).wait()

    return gather_cols(table_t, ids)


def kernel(input_ids, input_states, lora_A, lora_B_w):
    b, s = input_ids.shape
    h = input_states.shape[-1]
    r = lora_A.shape[1]
    n = b * s

    ids = input_ids.reshape(n).astype(jnp.int32)
    gathered_t = _sc_gather_t(lora_A.T, ids, n, r)

    x2d = input_states.reshape(n, h)
    w_t = lora_B_w.T
    blk = 512

    def tc_body(g_ref, x_ref, w_ref, o_ref):
        prj = lax.dot_general(
            g_ref[...],
            w_ref[...],
            dimension_numbers=(((0,), (0,)), ((), ())),
            preferred_element_type=jnp.float32,
        )
        o_ref[...] = x_ref[...] + prj

    out2d = pl.pallas_call(
        tc_body,
        grid=(n // blk,),
        in_specs=[
            pl.BlockSpec((r, blk), lambda i: (0, i)),
            pl.BlockSpec((blk, h), lambda i: (i, 0)),
            pl.BlockSpec((r, h), lambda i: (0, 0)),
        ],
        out_specs=pl.BlockSpec((blk, h), lambda i: (i, 0)),
        out_shape=jax.ShapeDtypeStruct((n, h), jnp.float32),
    )(gathered_t, x2d, w_t)

    return out2d.reshape(b, s, h)
